# Initial kernel scaffold; baseline (speedup 1.0000x reference)
import jax
import jax.numpy as jnp
from jax.experimental import pallas as pl


def kernel(x_author, x_paper, edge_index_writes, edge_index_written_by, Wl1_w, Wr1_w, b1_w, Wl1_wb, Wr1_wb, b1_wb, Wl2_w, Wr2_w, b2_w, Wl2_wb, Wr2_wb, b2_wb, skipA_W, skipA_b, skipP_W, skipP_b, cls1_W, cls1_b, cls2_W, cls2_b):
    raise NotImplementedError("write your pallas kernel here")


# speedup vs baseline: 4.3559x; 4.3559x over previous
import jax
import jax.numpy as jnp
from jax import lax
from jax.experimental import pallas as pl
from jax.experimental.pallas import tpu as pltpu
from jax.experimental.pallas import tpu_sc as plsc

F = 128
FA = 144
NS = 16
CHUNK = 80


def _make_spmm_pair(N, E, fw):
    EPT = E // NS
    NCH = EPT // CHUNK
    RPS = 632
    RLAST = N - (NS - 1) * RPS

    mesh = plsc.VectorSubcoreMesh(core_axis_name="c", subcore_axis_name="s")

    out_type = (jax.ShapeDtypeStruct((N, fw), jnp.float32),
                jax.ShapeDtypeStruct((N, fw), jnp.float32))
    scratch = (
        pltpu.VMEM_SHARED((N, fw), jnp.float32),
        pltpu.VMEM((CHUNK,), jnp.int32),
        pltpu.VMEM((CHUNK,), jnp.int32),
        pltpu.VMEM((CHUNK, fw), jnp.float32),
        pltpu.SemaphoreType.DMA,
    )

    def body(xa_h, xp_h, s0_h, d0_h, s1_h, d1_h, z_h, o0_h, o1_h,
             acc, sv, dv, rows, sem):
        cid = lax.axis_index("c")
        sid = lax.axis_index("s")
        row0 = sid * RPS

        pltpu.sync_copy(z_h, rows)

        def init_slice(nrows):
            nfull = nrows // CHUNK
            rem = nrows - nfull * CHUNK
            for j in range(nfull):
                pltpu.sync_copy(rows, acc.at[pl.ds(row0 + j * CHUNK, CHUNK)])
            if rem:
                pltpu.sync_copy(rows.at[pl.ds(0, rem)],
                                acc.at[pl.ds(row0 + nfull * CHUNK, rem)])

        pl.when(sid < NS - 1)(lambda: init_slice(RPS))
        pl.when(sid == NS - 1)(lambda: init_slice(RLAST))
        plsc.subcore_barrier()

        def process(x_h, s_h, d_h, o_h):
            base = sid * EPT

            def ch(k, c):
                off = base + k * CHUNK
                pltpu.sync_copy(s_h.at[pl.ds(off, CHUNK)], sv)
                pltpu.sync_copy(d_h.at[pl.ds(off, CHUNK)], dv)
                pltpu.async_copy(x_h.at[sv], rows, sem).wait()
                pltpu.sync_copy(rows, acc.at[dv], add=True)
                return c

            lax.fori_loop(0, NCH, ch, 0)
            plsc.subcore_barrier()

            def readback(nrows):
                nfull = nrows // CHUNK
                rem = nrows - nfull * CHUNK
                for j in range(nfull + (1 if rem else 0)):
                    n = CHUNK if j < nfull else rem
                    r0 = row0 + j * CHUNK
                    pltpu.sync_copy(acc.at[pl.ds(r0, n)], rows.at[pl.ds(0, n)])
                    pltpu.sync_copy(rows.at[pl.ds(0, n)], o_h.at[pl.ds(r0, n)])

            pl.when(sid < NS - 1)(lambda: readback(RPS))
            pl.when(sid == NS - 1)(lambda: readback(RLAST))

        pl.when(cid == 0)(lambda: process(xa_h, s0_h, d0_h, o0_h))
        pl.when(cid == 1)(lambda: process(xp_h, s1_h, d1_h, o1_h))

    return pl.kernel(body, out_type=out_type, mesh=mesh, scratch_types=scratch,
                     compiler_params=pltpu.CompilerParams(use_tc_tiling_on_sc=False))


def _relu(x):
    return jnp.maximum(x, 0.0)


def _dot(a, b):
    return jnp.dot(a, b, preferred_element_type=jnp.float32)


def _tc1(o0, o1, xa, xp, wl1w, wr1w, b1w, wl1wb, wr1wb, b1wb):
    N = xa.shape[0]
    B = 1000
    G = N // B

    def body(o0_r, o1_r, xa_r, xp_r, wl1w_r, wr1w_r, b1w_r,
             wl1wb_r, wr1wb_r, b1wb_r, h1p_o, h1a_o, c0_o, c1_o):
        a0 = o0_r[...]
        a1 = o1_r[...]
        c0 = jnp.maximum(a0[:, F:F + 1], 1.0)
        c1 = jnp.maximum(a1[:, F:F + 1], 1.0)
        meanp = a0[:, 0:F] / c0
        meana = a1[:, 0:F] / c1
        h1p_o[...] = _relu(_dot(meanp, wl1w_r[...]) + _dot(xp_r[...], wr1w_r[...])
                           + b1w_r[...])
        h1a_o[...] = _relu(_dot(meana, wl1wb_r[...]) + _dot(xa_r[...], wr1wb_r[...])
                           + b1wb_r[...])
        c0_o[...] = jnp.broadcast_to(c0, (B, 16))
        c1_o[...] = jnp.broadcast_to(c1, (B, 16))

    arow_spec = pl.BlockSpec((B, FA), lambda i: (i, 0))
    row_spec = pl.BlockSpec((B, F), lambda i: (i, 0))
    cnt_spec = pl.BlockSpec((B, 16), lambda i: (i, 0))
    w_spec = pl.BlockSpec((F, F), lambda i: (0, 0))
    b_spec = pl.BlockSpec((1, F), lambda i: (0, 0))
    return pl.pallas_call(
        body,
        grid=(G,),
        in_specs=[arow_spec, arow_spec, row_spec, row_spec,
                  w_spec, w_spec, b_spec, w_spec, w_spec, b_spec],
        out_specs=[row_spec, row_spec, cnt_spec, cnt_spec],
        out_shape=[jax.ShapeDtypeStruct((N, F), jnp.float32),
                   jax.ShapeDtypeStruct((N, F), jnp.float32),
                   jax.ShapeDtypeStruct((N, 16), jnp.float32),
                   jax.ShapeDtypeStruct((N, 16), jnp.float32)],
    )(o0, o1, xa, xp, wl1w, wr1w, b1w, wl1wb, wr1wb, b1wb)


def _tc2(s0, s1, c0x, c1x, h1p, h1a, xa, xp,
         wl2w, wr2w, b2w, wl2wb, wr2wb, b2wb,
         skA, skAb, skP, skPb, c1W, c1b, c2W, c2b):
    N = xa.shape[0]
    B = 1000
    G = N // B

    def body(s0_r, s1_r, c0_r, c1_r, h1p_r, h1a_r, xa_r, xp_r,
             wl2w_r, wr2w_r, b2w_r, wl2wb_r, wr2wb_r, b2wb_r,
             skA_r, skAb_r, skP_r, skPb_r, c1W_r, c1b_r, c2W_r, c2b_r,
             out_o, accP, accA):
        i = pl.program_id(0)
        c0 = c0_r[...][:, 0:1]
        c1 = c1_r[...][:, 0:1]
        h2p = (_dot(s0_r[...] / c0, wl2w_r[...])
               + _dot(h1p_r[...], wr2w_r[...]) + b2w_r[...])
        h2a = (_dot(s1_r[...] / c1, wl2wb_r[...])
               + _dot(h1a_r[...], wr2wb_r[...]) + b2wb_r[...])
        h2p = _relu(h2p + _dot(xp_r[...], skP_r[...]) + skPb_r[...])
        h2a = _relu(h2a + _dot(xa_r[...], skA_r[...]) + skAb_r[...])
        pp = jnp.broadcast_to(jnp.sum(h2p, axis=0, keepdims=True), (8, F))
        pa = jnp.broadcast_to(jnp.sum(h2a, axis=0, keepdims=True), (8, F))

        @pl.when(i == 0)
        def _():
            accP[...] = pp
            accA[...] = pa

        @pl.when(i > 0)
        def _():
            accP[...] += pp
            accA[...] += pa

        @pl.when(i == G - 1)
        def _():
            inv_n = 1.0 / N
            apool = accA[...] * inv_n
            ppool = accP[...] * inv_n
            h = _relu(_dot(apool[0:1, :], c1W_r[0:F, :])
                      + _dot(ppool[0:1, :], c1W_r[F:2 * F, :]) + c1b_r[...])
            o = _dot(h, c2W_r[...]) + c2b_r[...]
            out_o[...] = jnp.broadcast_to(o, (8, F))

    row_spec = pl.BlockSpec((B, F), lambda i: (i, 0))
    cnt_spec = pl.BlockSpec((B, 16), lambda i: (i, 0))
    w_spec = pl.BlockSpec((F, F), lambda i: (0, 0))
    b_spec = pl.BlockSpec((1, F), lambda i: (0, 0))
    c1w_spec = pl.BlockSpec((2 * F, F), lambda i: (0, 0))
    out_spec = pl.BlockSpec((8, F), lambda i: (0, 0))
    return pl.pallas_call(
        body,
        grid=(G,),
        in_specs=[row_spec, row_spec, cnt_spec, cnt_spec, row_spec, row_spec,
                  row_spec, row_spec,
                  w_spec, w_spec, b_spec, w_spec, w_spec, b_spec,
                  w_spec, b_spec, w_spec, b_spec, c1w_spec, b_spec,
                  w_spec, b_spec],
        out_specs=[out_spec],
        out_shape=[jax.ShapeDtypeStruct((8, F), jnp.float32)],
        scratch_shapes=[pltpu.VMEM((8, F), jnp.float32),
                        pltpu.VMEM((8, F), jnp.float32)],
    )(s0, s1, c0x, c1x, h1p, h1a, xa, xp,
      wl2w, wr2w, b2w, wl2wb, wr2wb, b2wb,
      skA, skAb, skP, skPb, c1W, c1b, c2W, c2b)[0]


def kernel(x_author, x_paper, edge_index_writes, edge_index_written_by,
           Wl1_w, Wr1_w, b1_w, Wl1_wb, Wr1_wb, b1_wb,
           Wl2_w, Wr2_w, b2_w, Wl2_wb, Wr2_wb, b2_wb,
           skipA_W, skipA_b, skipP_W, skipP_b,
           cls1_W, cls1_b, cls2_W, cls2_b):
    N = x_author.shape[0]
    E = edge_index_writes.shape[1]
    src_w = edge_index_writes[0]
    dst_w = edge_index_writes[1]
    src_wb = edge_index_written_by[0]
    dst_wb = edge_index_written_by[1]
    zeros_a = jnp.zeros((CHUNK, FA), jnp.float32)
    zeros_f = jnp.zeros((CHUNK, F), jnp.float32)

    def aug(x):
        return jnp.concatenate(
            [x, jnp.ones((N, 1), jnp.float32), jnp.zeros((N, FA - F - 1), jnp.float32)],
            axis=1)

    spmm1 = _make_spmm_pair(N, E, FA)
    spmm2 = _make_spmm_pair(N, E, F)

    o1p, o1a = spmm1(aug(x_author), aug(x_paper), src_w, dst_w, src_wb, dst_wb,
                     zeros_a)

    h1p, h1a, c0x, c1x = _tc1(o1p, o1a, x_author, x_paper,
                              Wl1_w, Wr1_w, b1_w.reshape(1, F),
                              Wl1_wb, Wr1_wb, b1_wb.reshape(1, F))

    s2p, s2a = spmm2(h1a, h1p, src_w, dst_w, src_wb, dst_wb, zeros_f)

    OUT = cls2_W.shape[1]
    c2Wp = jnp.pad(cls2_W, ((0, 0), (0, F - OUT)))
    c2bp = jnp.pad(cls2_b, (0, F - OUT)).reshape(1, F)
    outp = _tc2(s2p, s2a, c0x, c1x, h1p, h1a, x_author, x_paper,
                Wl2_w, Wr2_w, b2_w.reshape(1, F),
                Wl2_wb, Wr2_wb, b2_wb.reshape(1, F),
                skipA_W, skipA_b.reshape(1, F), skipP_W, skipP_b.reshape(1, F),
                cls1_W, cls1_b.reshape(1, F), c2Wp, c2bp)
    return outp[0:1, 0:OUT]


# speedup vs baseline: 7.9233x; 1.8190x over previous
import jax
import jax.numpy as jnp
from jax import lax
from jax.experimental import pallas as pl
from jax.experimental.pallas import tpu as pltpu
from jax.experimental.pallas import tpu_sc as plsc

F = 128
FA = 144
NS = 16
CHUNK = 80


def _make_spmm_pair(N, E, fw):
    NCH = E // CHUNK // NS
    IBLK = 50
    NBLK = NCH // IBLK
    RPS = 632
    RLAST = N - (NS - 1) * RPS

    mesh = plsc.VectorSubcoreMesh(core_axis_name="c", subcore_axis_name="s")

    out_type = (jax.ShapeDtypeStruct((N, fw), jnp.float32),
                jax.ShapeDtypeStruct((N, fw), jnp.float32))
    scratch = (
        pltpu.VMEM_SHARED((N, fw), jnp.float32),
        pltpu.VMEM((IBLK, CHUNK), jnp.int32),
        pltpu.VMEM((IBLK, CHUNK), jnp.int32),
        pltpu.VMEM((CHUNK, fw), jnp.float32),
        pltpu.VMEM((CHUNK, fw), jnp.float32),
        pltpu.SemaphoreType.DMA,
        pltpu.SemaphoreType.DMA,
    )

    def body(xa_h, xp_h, s0_h, d0_h, s1_h, d1_h, z_h, o0_h, o1_h,
             acc, sv, dv, rows0, rows1, gsem, ssem):
        cid = lax.axis_index("c")
        sid = lax.axis_index("s")
        row0 = sid * RPS

        pltpu.sync_copy(z_h, rows0)

        def init_slice(nrows):
            nfull = nrows // CHUNK
            rem = nrows - nfull * CHUNK
            for j in range(nfull):
                pltpu.sync_copy(rows0, acc.at[pl.ds(row0 + j * CHUNK, CHUNK)])
            if rem:
                pltpu.sync_copy(rows0.at[pl.ds(0, rem)],
                                acc.at[pl.ds(row0 + nfull * CHUNK, rem)])

        pl.when(sid < NS - 1)(lambda: init_slice(RPS))
        pl.when(sid == NS - 1)(lambda: init_slice(RLAST))
        plsc.subcore_barrier()

        def process(x_h, s_h, d_h, o_h):
            base = sid * NCH

            def wait_g(buf):
                pltpu.make_async_copy(x_h.at[pl.ds(0, CHUNK)], buf, gsem).wait()

            def wait_s(buf):
                pltpu.make_async_copy(x_h.at[pl.ds(0, CHUNK)], buf, ssem).wait()

            def gather(row, buf):
                pltpu.async_copy(x_h.at[sv.at[row]], buf, gsem)

            def scatter(row, buf):
                pltpu.async_copy(buf, acc.at[dv.at[row]], ssem, add=True)

            def pair(a, prefetch):
                wait_g(rows0)
                scatter(a, rows0)
                gather(a + 1, rows1)
                wait_g(rows1)
                wait_s(rows0)
                scatter(a + 1, rows1)
                if prefetch is not None:
                    gather(prefetch, rows0)
                wait_s(rows1)

            for m in range(NBLK):
                pltpu.sync_copy(s_h.at[pl.ds(base + m * IBLK, IBLK)], sv)
                pltpu.sync_copy(d_h.at[pl.ds(base + m * IBLK, IBLK)], dv)
                gather(0, rows0)

                def steady(j, c):
                    pair(2 * j, 2 * j + 2)
                    return c

                lax.fori_loop(0, IBLK // 2 - 1, steady, 0)
                pair(IBLK - 2, None)

            plsc.subcore_barrier()

            def readback(nrows):
                nfull = nrows // CHUNK
                rem = nrows - nfull * CHUNK
                steps = nfull + (1 if rem else 0)

                def geom(j):
                    n = CHUNK if j < nfull else rem
                    return n, row0 + j * CHUNK, (rows0 if j % 2 == 0 else rows1)

                def wait_out(j):
                    n, r0, buf = geom(j)
                    pltpu.make_async_copy(buf.at[pl.ds(0, n)],
                                          o_h.at[pl.ds(r0, n)], gsem).wait()

                for j in range(steps):
                    n, r0, buf = geom(j)
                    if j >= 2:
                        wait_out(j - 2)
                    pltpu.sync_copy(acc.at[pl.ds(r0, n)], buf.at[pl.ds(0, n)])
                    pltpu.async_copy(buf.at[pl.ds(0, n)],
                                     o_h.at[pl.ds(r0, n)], gsem)
                for j in range(max(0, steps - 2), steps):
                    wait_out(j)

            pl.when(sid < NS - 1)(lambda: readback(RPS))
            pl.when(sid == NS - 1)(lambda: readback(RLAST))

        pl.when(cid == 0)(lambda: process(xa_h, s0_h, d0_h, o0_h))
        pl.when(cid == 1)(lambda: process(xp_h, s1_h, d1_h, o1_h))

    return pl.kernel(body, out_type=out_type, mesh=mesh, scratch_types=scratch,
                     compiler_params=pltpu.CompilerParams(use_tc_tiling_on_sc=False))


def _relu(x):
    return jnp.maximum(x, 0.0)


def _dot(a, b):
    return jnp.dot(a, b, preferred_element_type=jnp.float32)


def _tc1(o0, o1, xa, xp, wl1w, wr1w, b1w, wl1wb, wr1wb, b1wb):
    N = xa.shape[0]
    B = 1000
    G = N // B

    def body(o0_r, o1_r, xa_r, xp_r, wl1w_r, wr1w_r, b1w_r,
             wl1wb_r, wr1wb_r, b1wb_r, h1p_o, h1a_o, c0_o, c1_o):
        a0 = o0_r[...]
        a1 = o1_r[...]
        c0 = jnp.maximum(a0[:, F:F + 1], 1.0)
        c1 = jnp.maximum(a1[:, F:F + 1], 1.0)
        meanp = a0[:, 0:F] / c0
        meana = a1[:, 0:F] / c1
        h1p_o[...] = _relu(_dot(meanp, wl1w_r[...]) + _dot(xp_r[...], wr1w_r[...])
                           + b1w_r[...])
        h1a_o[...] = _relu(_dot(meana, wl1wb_r[...]) + _dot(xa_r[...], wr1wb_r[...])
                           + b1wb_r[...])
        c0_o[...] = jnp.broadcast_to(c0, (B, 16))
        c1_o[...] = jnp.broadcast_to(c1, (B, 16))

    arow_spec = pl.BlockSpec((B, FA), lambda i: (i, 0))
    row_spec = pl.BlockSpec((B, F), lambda i: (i, 0))
    cnt_spec = pl.BlockSpec((B, 16), lambda i: (i, 0))
    w_spec = pl.BlockSpec((F, F), lambda i: (0, 0))
    b_spec = pl.BlockSpec((1, F), lambda i: (0, 0))
    return pl.pallas_call(
        body,
        grid=(G,),
        in_specs=[arow_spec, arow_spec, row_spec, row_spec,
                  w_spec, w_spec, b_spec, w_spec, w_spec, b_spec],
        out_specs=[row_spec, row_spec, cnt_spec, cnt_spec],
        out_shape=[jax.ShapeDtypeStruct((N, F), jnp.float32),
                   jax.ShapeDtypeStruct((N, F), jnp.float32),
                   jax.ShapeDtypeStruct((N, 16), jnp.float32),
                   jax.ShapeDtypeStruct((N, 16), jnp.float32)],
    )(o0, o1, xa, xp, wl1w, wr1w, b1w, wl1wb, wr1wb, b1wb)


def _tc2(s0, s1, c0x, c1x, h1p, h1a, xa, xp,
         wl2w, wr2w, b2w, wl2wb, wr2wb, b2wb,
         skA, skAb, skP, skPb, c1W, c1b, c2W, c2b):
    N = xa.shape[0]
    B = 1000
    G = N // B

    def body(s0_r, s1_r, c0_r, c1_r, h1p_r, h1a_r, xa_r, xp_r,
             wl2w_r, wr2w_r, b2w_r, wl2wb_r, wr2wb_r, b2wb_r,
             skA_r, skAb_r, skP_r, skPb_r, c1W_r, c1b_r, c2W_r, c2b_r,
             out_o, accP, accA):
        i = pl.program_id(0)
        c0 = c0_r[...][:, 0:1]
        c1 = c1_r[...][:, 0:1]
        h2p = (_dot(s0_r[...] / c0, wl2w_r[...])
               + _dot(h1p_r[...], wr2w_r[...]) + b2w_r[...])
        h2a = (_dot(s1_r[...] / c1, wl2wb_r[...])
               + _dot(h1a_r[...], wr2wb_r[...]) + b2wb_r[...])
        h2p = _relu(h2p + _dot(xp_r[...], skP_r[...]) + skPb_r[...])
        h2a = _relu(h2a + _dot(xa_r[...], skA_r[...]) + skAb_r[...])
        pp = jnp.broadcast_to(jnp.sum(h2p, axis=0, keepdims=True), (8, F))
        pa = jnp.broadcast_to(jnp.sum(h2a, axis=0, keepdims=True), (8, F))

        @pl.when(i == 0)
        def _():
            accP[...] = pp
            accA[...] = pa

        @pl.when(i > 0)
        def _():
            accP[...] += pp
            accA[...] += pa

        @pl.when(i == G - 1)
        def _():
            inv_n = 1.0 / N
            apool = accA[...] * inv_n
            ppool = accP[...] * inv_n
            h = _relu(_dot(apool[0:1, :], c1W_r[0:F, :])
                      + _dot(ppool[0:1, :], c1W_r[F:2 * F, :]) + c1b_r[...])
            o = _dot(h, c2W_r[...]) + c2b_r[...]
            out_o[...] = jnp.broadcast_to(o, (8, F))

    row_spec = pl.BlockSpec((B, F), lambda i: (i, 0))
    cnt_spec = pl.BlockSpec((B, 16), lambda i: (i, 0))
    w_spec = pl.BlockSpec((F, F), lambda i: (0, 0))
    b_spec = pl.BlockSpec((1, F), lambda i: (0, 0))
    c1w_spec = pl.BlockSpec((2 * F, F), lambda i: (0, 0))
    out_spec = pl.BlockSpec((8, F), lambda i: (0, 0))
    return pl.pallas_call(
        body,
        grid=(G,),
        in_specs=[row_spec, row_spec, cnt_spec, cnt_spec, row_spec, row_spec,
                  row_spec, row_spec,
                  w_spec, w_spec, b_spec, w_spec, w_spec, b_spec,
                  w_spec, b_spec, w_spec, b_spec, c1w_spec, b_spec,
                  w_spec, b_spec],
        out_specs=[out_spec],
        out_shape=[jax.ShapeDtypeStruct((8, F), jnp.float32)],
        scratch_shapes=[pltpu.VMEM((8, F), jnp.float32),
                        pltpu.VMEM((8, F), jnp.float32)],
    )(s0, s1, c0x, c1x, h1p, h1a, xa, xp,
      wl2w, wr2w, b2w, wl2wb, wr2wb, b2wb,
      skA, skAb, skP, skPb, c1W, c1b, c2W, c2b)[0]


def kernel(x_author, x_paper, edge_index_writes, edge_index_written_by,
           Wl1_w, Wr1_w, b1_w, Wl1_wb, Wr1_wb, b1_wb,
           Wl2_w, Wr2_w, b2_w, Wl2_wb, Wr2_wb, b2_wb,
           skipA_W, skipA_b, skipP_W, skipP_b,
           cls1_W, cls1_b, cls2_W, cls2_b):
    N = x_author.shape[0]
    E = edge_index_writes.shape[1]
    src_w = edge_index_writes[0].reshape(-1, CHUNK)
    dst_w = edge_index_writes[1].reshape(-1, CHUNK)
    src_wb = edge_index_written_by[0].reshape(-1, CHUNK)
    dst_wb = edge_index_written_by[1].reshape(-1, CHUNK)
    zeros_a = jnp.zeros((CHUNK, FA), jnp.float32)
    zeros_f = jnp.zeros((CHUNK, F), jnp.float32)

    def aug(x):
        return jnp.concatenate(
            [x, jnp.ones((N, 1), jnp.float32), jnp.zeros((N, FA - F - 1), jnp.float32)],
            axis=1)

    spmm1 = _make_spmm_pair(N, E, FA)
    spmm2 = _make_spmm_pair(N, E, F)

    o1p, o1a = spmm1(aug(x_author), aug(x_paper), src_w, dst_w, src_wb, dst_wb,
                     zeros_a)

    h1p, h1a, c0x, c1x = _tc1(o1p, o1a, x_author, x_paper,
                              Wl1_w, Wr1_w, b1_w.reshape(1, F),
                              Wl1_wb, Wr1_wb, b1_wb.reshape(1, F))

    s2p, s2a = spmm2(h1a, h1p, src_w, dst_w, src_wb, dst_wb, zeros_f)

    OUT = cls2_W.shape[1]
    c2Wp = jnp.pad(cls2_W, ((0, 0), (0, F - OUT)))
    c2bp = jnp.pad(cls2_b, (0, F - OUT)).reshape(1, F)
    outp = _tc2(s2p, s2a, c0x, c1x, h1p, h1a, x_author, x_paper,
                Wl2_w, Wr2_w, b2_w.reshape(1, F),
                Wl2_wb, Wr2_wb, b2_wb.reshape(1, F),
                skipA_W, skipA_b.reshape(1, F), skipP_W, skipP_b.reshape(1, F),
                cls1_W, cls1_b.reshape(1, F), c2Wp, c2bp)
    return outp[0:1, 0:OUT]


# speedup vs baseline: 10.7580x; 1.3578x over previous
import jax
import jax.numpy as jnp
from jax import lax
from jax.experimental import pallas as pl
from jax.experimental.pallas import tpu as pltpu
from jax.experimental.pallas import tpu_sc as plsc

F = 128
FA = 144
NS = 16
CHUNK = 80


def _make_spmm_pair(N, E, fw):
    NCH = E // CHUNK // NS
    IBLK = 25
    NBLK = NCH // IBLK
    NB = 3
    RPS = 632
    RLAST = N - (NS - 1) * RPS

    mesh = plsc.VectorSubcoreMesh(core_axis_name="c", subcore_axis_name="s")

    out_type = (jax.ShapeDtypeStruct((N, fw), jnp.float32),
                jax.ShapeDtypeStruct((N, fw), jnp.float32))
    scratch = (
        pltpu.VMEM_SHARED((N, fw), jnp.float32),
        pltpu.VMEM((IBLK, CHUNK), jnp.int32),
        pltpu.VMEM((IBLK, CHUNK), jnp.int32),
        pltpu.VMEM((CHUNK, fw), jnp.float32),
        pltpu.VMEM((CHUNK, fw), jnp.float32),
        pltpu.VMEM((CHUNK, fw), jnp.float32),
        pltpu.SemaphoreType.DMA,
        pltpu.SemaphoreType.DMA,
    )

    def body(xa_h, xp_h, s0_h, d0_h, s1_h, d1_h, z_h, o0_h, o1_h,
             acc, sv, dv, rows0, rows1, rows2, gsem, ssem):
        bufs = (rows0, rows1, rows2)
        cid = lax.axis_index("c")
        sid = lax.axis_index("s")
        row0 = sid * RPS

        pltpu.sync_copy(z_h, rows0)

        def init_slice(nrows):
            nfull = nrows // CHUNK
            rem = nrows - nfull * CHUNK
            for j in range(nfull):
                pltpu.sync_copy(rows0, acc.at[pl.ds(row0 + j * CHUNK, CHUNK)])
            if rem:
                pltpu.sync_copy(rows0.at[pl.ds(0, rem)],
                                acc.at[pl.ds(row0 + nfull * CHUNK, rem)])

        pl.when(sid < NS - 1)(lambda: init_slice(RPS))
        pl.when(sid == NS - 1)(lambda: init_slice(RLAST))
        plsc.subcore_barrier()

        def process(x_h, s_h, d_h, o_h):
            base = sid * NCH

            def wait_g(buf):
                pltpu.make_async_copy(x_h.at[pl.ds(0, CHUNK)], buf, gsem).wait()

            def wait_s(buf):
                pltpu.make_async_copy(x_h.at[pl.ds(0, CHUNK)], buf, ssem).wait()

            def gather(row, buf):
                pltpu.async_copy(x_h.at[sv.at[row]], buf, gsem)

            def scatter(row, buf):
                pltpu.async_copy(buf, acc.at[dv.at[row]], ssem, add=True)

            def step(k, phase, wait_prev, prefetch):
                wait_g(bufs[phase])
                scatter(k, bufs[phase])
                if wait_prev:
                    wait_s(bufs[(phase + 2) % NB])
                if prefetch:
                    gather(k + 2, bufs[(phase + 2) % NB])

            def block(m, c):
                pltpu.sync_copy(s_h.at[pl.ds(base + m * IBLK, IBLK)], sv)
                pltpu.sync_copy(d_h.at[pl.ds(base + m * IBLK, IBLK)], dv)
                gather(0, bufs[0])
                gather(1, bufs[1])
                step(0, 0, False, True)
                step(1, 1, True, True)
                step(2, 2, True, True)

                def steady(j, cc):
                    k = 3 * j
                    step(k, 0, True, True)
                    step(k + 1, 1, True, True)
                    step(k + 2, 2, True, True)
                    return cc

                lax.fori_loop(1, (IBLK - 4) // 3, steady, 0)
                step(IBLK - 4, 0, True, True)
                step(IBLK - 3, 1, True, True)
                step(IBLK - 2, 2, True, False)
                step(IBLK - 1, 0, True, False)
                wait_s(bufs[0])
                return c

            lax.fori_loop(0, NBLK, block, 0)

            plsc.subcore_barrier()

            def readback(nrows):
                nfull = nrows // CHUNK
                rem = nrows - nfull * CHUNK
                steps = nfull + (1 if rem else 0)

                def geom(j):
                    n = CHUNK if j < nfull else rem
                    return n, row0 + j * CHUNK, (rows0 if j % 2 == 0 else rows1)

                def wait_out(j):
                    n, r0, buf = geom(j)
                    pltpu.make_async_copy(buf.at[pl.ds(0, n)],
                                          o_h.at[pl.ds(r0, n)], gsem).wait()

                for j in range(steps):
                    n, r0, buf = geom(j)
                    if j >= 2:
                        wait_out(j - 2)
                    pltpu.sync_copy(acc.at[pl.ds(r0, n)], buf.at[pl.ds(0, n)])
                    pltpu.async_copy(buf.at[pl.ds(0, n)],
                                     o_h.at[pl.ds(r0, n)], gsem)
                for j in range(max(0, steps - 2), steps):
                    wait_out(j)

            pl.when(sid < NS - 1)(lambda: readback(RPS))
            pl.when(sid == NS - 1)(lambda: readback(RLAST))

        pl.when(cid == 0)(lambda: process(xa_h, s0_h, d0_h, o0_h))
        pl.when(cid == 1)(lambda: process(xp_h, s1_h, d1_h, o1_h))

    return pl.kernel(body, out_type=out_type, mesh=mesh, scratch_types=scratch,
                     compiler_params=pltpu.CompilerParams(use_tc_tiling_on_sc=False))


def _relu(x):
    return jnp.maximum(x, 0.0)


def _dot(a, b):
    return jnp.dot(a, b, preferred_element_type=jnp.float32)


def _tc1(o0, o1, xa, xp, wl1w, wr1w, b1w, wl1wb, wr1wb, b1wb):
    N = xa.shape[0]
    B = 1000
    G = N // B

    def body(o0_r, o1_r, xa_r, xp_r, wl1w_r, wr1w_r, b1w_r,
             wl1wb_r, wr1wb_r, b1wb_r, h1p_o, h1a_o, c0_o, c1_o):
        a0 = o0_r[...]
        a1 = o1_r[...]
        c0 = jnp.maximum(a0[:, F:F + 1], 1.0)
        c1 = jnp.maximum(a1[:, F:F + 1], 1.0)
        meanp = a0[:, 0:F] / c0
        meana = a1[:, 0:F] / c1
        h1p_o[...] = _relu(_dot(meanp, wl1w_r[...]) + _dot(xp_r[...], wr1w_r[...])
                           + b1w_r[...])
        h1a_o[...] = _relu(_dot(meana, wl1wb_r[...]) + _dot(xa_r[...], wr1wb_r[...])
                           + b1wb_r[...])
        c0_o[...] = jnp.broadcast_to(c0, (B, 16))
        c1_o[...] = jnp.broadcast_to(c1, (B, 16))

    arow_spec = pl.BlockSpec((B, FA), lambda i: (i, 0))
    row_spec = pl.BlockSpec((B, F), lambda i: (i, 0))
    cnt_spec = pl.BlockSpec((B, 16), lambda i: (i, 0))
    w_spec = pl.BlockSpec((F, F), lambda i: (0, 0))
    b_spec = pl.BlockSpec((1, F), lambda i: (0, 0))
    return pl.pallas_call(
        body,
        grid=(G,),
        in_specs=[arow_spec, arow_spec, row_spec, row_spec,
                  w_spec, w_spec, b_spec, w_spec, w_spec, b_spec],
        out_specs=[row_spec, row_spec, cnt_spec, cnt_spec],
        out_shape=[jax.ShapeDtypeStruct((N, F), jnp.float32),
                   jax.ShapeDtypeStruct((N, F), jnp.float32),
                   jax.ShapeDtypeStruct((N, 16), jnp.float32),
                   jax.ShapeDtypeStruct((N, 16), jnp.float32)],
    )(o0, o1, xa, xp, wl1w, wr1w, b1w, wl1wb, wr1wb, b1wb)


def _tc2(s0, s1, c0x, c1x, h1p, h1a, xa, xp,
         wl2w, wr2w, b2w, wl2wb, wr2wb, b2wb,
         skA, skAb, skP, skPb, c1W, c1b, c2W, c2b):
    N = xa.shape[0]
    B = 1000
    G = N // B

    def body(s0_r, s1_r, c0_r, c1_r, h1p_r, h1a_r, xa_r, xp_r,
             wl2w_r, wr2w_r, b2w_r, wl2wb_r, wr2wb_r, b2wb_r,
             skA_r, skAb_r, skP_r, skPb_r, c1W_r, c1b_r, c2W_r, c2b_r,
             out_o, accP, accA):
        i = pl.program_id(0)
        c0 = c0_r[...][:, 0:1]
        c1 = c1_r[...][:, 0:1]
        h2p = (_dot(s0_r[...] / c0, wl2w_r[...])
               + _dot(h1p_r[...], wr2w_r[...]) + b2w_r[...])
        h2a = (_dot(s1_r[...] / c1, wl2wb_r[...])
               + _dot(h1a_r[...], wr2wb_r[...]) + b2wb_r[...])
        h2p = _relu(h2p + _dot(xp_r[...], skP_r[...]) + skPb_r[...])
        h2a = _relu(h2a + _dot(xa_r[...], skA_r[...]) + skAb_r[...])
        pp = jnp.broadcast_to(jnp.sum(h2p, axis=0, keepdims=True), (8, F))
        pa = jnp.broadcast_to(jnp.sum(h2a, axis=0, keepdims=True), (8, F))

        @pl.when(i == 0)
        def _():
            accP[...] = pp
            accA[...] = pa

        @pl.when(i > 0)
        def _():
            accP[...] += pp
            accA[...] += pa

        @pl.when(i == G - 1)
        def _():
            inv_n = 1.0 / N
            apool = accA[...] * inv_n
            ppool = accP[...] * inv_n
            h = _relu(_dot(apool[0:1, :], c1W_r[0:F, :])
                      + _dot(ppool[0:1, :], c1W_r[F:2 * F, :]) + c1b_r[...])
            o = _dot(h, c2W_r[...]) + c2b_r[...]
            out_o[...] = jnp.broadcast_to(o, (8, F))

    row_spec = pl.BlockSpec((B, F), lambda i: (i, 0))
    cnt_spec = pl.BlockSpec((B, 16), lambda i: (i, 0))
    w_spec = pl.BlockSpec((F, F), lambda i: (0, 0))
    b_spec = pl.BlockSpec((1, F), lambda i: (0, 0))
    c1w_spec = pl.BlockSpec((2 * F, F), lambda i: (0, 0))
    out_spec = pl.BlockSpec((8, F), lambda i: (0, 0))
    return pl.pallas_call(
        body,
        grid=(G,),
        in_specs=[row_spec, row_spec, cnt_spec, cnt_spec, row_spec, row_spec,
                  row_spec, row_spec,
                  w_spec, w_spec, b_spec, w_spec, w_spec, b_spec,
                  w_spec, b_spec, w_spec, b_spec, c1w_spec, b_spec,
                  w_spec, b_spec],
        out_specs=[out_spec],
        out_shape=[jax.ShapeDtypeStruct((8, F), jnp.float32)],
        scratch_shapes=[pltpu.VMEM((8, F), jnp.float32),
                        pltpu.VMEM((8, F), jnp.float32)],
    )(s0, s1, c0x, c1x, h1p, h1a, xa, xp,
      wl2w, wr2w, b2w, wl2wb, wr2wb, b2wb,
      skA, skAb, skP, skPb, c1W, c1b, c2W, c2b)[0]


def kernel(x_author, x_paper, edge_index_writes, edge_index_written_by,
           Wl1_w, Wr1_w, b1_w, Wl1_wb, Wr1_wb, b1_wb,
           Wl2_w, Wr2_w, b2_w, Wl2_wb, Wr2_wb, b2_wb,
           skipA_W, skipA_b, skipP_W, skipP_b,
           cls1_W, cls1_b, cls2_W, cls2_b):
    N = x_author.shape[0]
    E = edge_index_writes.shape[1]
    src_w = edge_index_writes[0].reshape(-1, CHUNK)
    dst_w = edge_index_writes[1].reshape(-1, CHUNK)
    src_wb = edge_index_written_by[0].reshape(-1, CHUNK)
    dst_wb = edge_index_written_by[1].reshape(-1, CHUNK)
    zeros_a = jnp.zeros((CHUNK, FA), jnp.float32)
    zeros_f = jnp.zeros((CHUNK, F), jnp.float32)

    def aug(x):
        return jnp.concatenate(
            [x, jnp.ones((N, 1), jnp.float32), jnp.zeros((N, FA - F - 1), jnp.float32)],
            axis=1)

    spmm1 = _make_spmm_pair(N, E, FA)
    spmm2 = _make_spmm_pair(N, E, F)

    o1p, o1a = spmm1(aug(x_author), aug(x_paper), src_w, dst_w, src_wb, dst_wb,
                     zeros_a)

    h1p, h1a, c0x, c1x = _tc1(o1p, o1a, x_author, x_paper,
                              Wl1_w, Wr1_w, b1_w.reshape(1, F),
                              Wl1_wb, Wr1_wb, b1_wb.reshape(1, F))

    s2p, s2a = spmm2(h1a, h1p, src_w, dst_w, src_wb, dst_wb, zeros_f)

    OUT = cls2_W.shape[1]
    c2Wp = jnp.pad(cls2_W, ((0, 0), (0, F - OUT)))
    c2bp = jnp.pad(cls2_b, (0, F - OUT)).reshape(1, F)
    outp = _tc2(s2p, s2a, c0x, c1x, h1p, h1a, x_author, x_paper,
                Wl2_w, Wr2_w, b2_w.reshape(1, F),
                Wl2_wb, Wr2_wb, b2_wb.reshape(1, F),
                skipA_W, skipA_b.reshape(1, F), skipP_W, skipP_b.reshape(1, F),
                cls1_W, cls1_b.reshape(1, F), c2Wp, c2bp)
    return outp[0:1, 0:OUT]
